# Initial kernel scaffold; baseline (speedup 1.0000x reference)
import jax
import jax.numpy as jnp
from jax.experimental import pallas as pl


def kernel(arr, ps):
    raise NotImplementedError("write your pallas kernel here")


# speedup vs baseline: 1.2157x; 1.2157x over previous
import functools

import jax
import jax.numpy as jnp
import numpy as np
from jax import lax
from jax.experimental import pallas as pl
from jax.experimental.pallas import tpu as pltpu
from jax.experimental.pallas import tpu_sc as plsc

_N = 16384
_LANES = 16
_M32 = 0xFFFFFFFF
_ROTS = ([13, 15, 26, 6], [17, 29, 16, 24])
_PARITY = 0x1BD11BDA


def _tf_host(k0, k1, x0, x1):
    ks2 = k0 ^ k1 ^ _PARITY
    inj = [(k1, (ks2 + 1) & _M32), (ks2, (k0 + 2) & _M32),
           (k0, (k1 + 3) & _M32), (k1, (ks2 + 4) & _M32),
           (ks2, (k0 + 5) & _M32)]
    a = (x0 + k0) & _M32
    b = (x1 + k1) & _M32
    for blk in range(5):
        for r in _ROTS[blk % 2]:
            a = (a + b) & _M32
            b = (((b << r) & _M32) | (b >> (32 - r))) ^ a
        pa, pb = inj[blk]
        a = (a + pa) & _M32
        b = (b + pb) & _M32
    return a, b


def _split_host(k0, k1):
    a0, b0 = _tf_host(k0, k1, 0, 0)
    a1, b1 = _tf_host(k0, k1, 0, 1)
    return (a0, b0), (a1, b1)


_KU, _KR = _split_host(0, 42)
_K1R, _K2R = _split_host(*_KR)
_KEYS = (_KU, _K1R, _K2R)


def _inj_consts(k0, k1):
    ks2 = k0 ^ k1 ^ _PARITY
    return [(k1, (ks2 + 1) & _M32), (ks2, (k0 + 2) & _M32),
            (k0, (k1 + 3) & _M32), (k1, (ks2 + 4) & _M32),
            (ks2, (k0 + 5) & _M32)]


_INJ = tuple(_inj_consts(k0, k1) for (k0, k1) in _KEYS)


def _tf3_bits(xlo):
    a = [jnp.full((_LANES,), np.uint32(k0), jnp.uint32) for (k0, _) in _KEYS]
    b = [xlo + np.uint32(k1) for (_, k1) in _KEYS]
    for blk in range(5):
        for r in _ROTS[blk % 2]:
            for t in range(3):
                a[t] = a[t] + b[t]
            for t in range(3):
                b[t] = ((b[t] << r) | (b[t] >> (32 - r))) ^ a[t]
        for t in range(3):
            pa, pb = _INJ[t][blk]
            a[t] = a[t] + np.uint32(pa)
            b[t] = b[t] + np.uint32(pb)
    return [a[t] ^ b[t] for t in range(3)]


_NC, _NS = 2, 16
_NW = _NC * _NS
_PER = _N // _NW
_CHUNKS = _PER // _LANES


def _rr_body(arr_hbm, ps_hbm, out_hbm, arr_v, ps_v, out_v):
    wid = lax.axis_index("s") * _NC + lax.axis_index("c")
    base = wid * _PER
    pltpu.sync_copy(arr_hbm.at[pl.ds(base, _PER)], arr_v)
    pltpu.sync_copy(ps_hbm, ps_v.at[pl.ds(0, 10)])
    ps_vec = ps_v[...]

    def body(j, carry):
        off = j * _LANES
        a_f = arr_v[pl.ds(off, _LANES)]
        idx = a_f.astype(jnp.int32)
        gidx = (lax.iota(jnp.int32, _LANES) + (base + off)).astype(jnp.uint32)
        bits_u, bits_h, bits_l = _tf3_bits(gidx)
        fbits = (bits_u >> np.uint32(9)) | np.uint32(0x3F800000)
        u = lax.bitcast_convert_type(fbits, jnp.float32) - jnp.float32(1.0)
        nine = np.uint32(9)
        ro = ((bits_h % nine) * np.uint32(4) + (bits_l % nine)) % nine
        r_i = ro.astype(jnp.int32)
        resp = jnp.where(r_i >= idx, r_i + jnp.int32(1), r_i)
        p_elem = jnp.take_along_axis(ps_vec, idx, axis=0)
        keep = u > p_elem
        out_v[pl.ds(off, _LANES)] = jnp.where(keep, a_f,
                                              resp.astype(jnp.float32))
        return carry

    lax.fori_loop(0, _CHUNKS, body, 0)
    pltpu.sync_copy(out_v, out_hbm.at[pl.ds(base, _PER)])


@functools.cache
def _build():
    mesh = plsc.VectorSubcoreMesh(core_axis_name="c", subcore_axis_name="s",
                                  num_cores=_NC, num_subcores=_NS)
    return pl.kernel(
        _rr_body,
        out_type=jax.ShapeDtypeStruct((_N,), jnp.float32),
        mesh=mesh,
        scratch_types=[
            pltpu.VMEM((_PER,), jnp.float32),
            pltpu.VMEM((_LANES,), jnp.float32),
            pltpu.VMEM((_PER,), jnp.float32),
        ],
    )


def kernel(arr, ps):
    return _build()(arr, ps)


# speedup vs baseline: 1.2445x; 1.0237x over previous
import functools

import jax
import jax.numpy as jnp
import numpy as np
from jax import lax
from jax.experimental import pallas as pl
from jax.experimental.pallas import tpu as pltpu
from jax.experimental.pallas import tpu_sc as plsc

_N = 16384
_LANES = 16
_M32 = 0xFFFFFFFF
_ROTS = ([13, 15, 26, 6], [17, 29, 16, 24])
_PARITY = 0x1BD11BDA


def _tf_host(k0, k1, x0, x1):
    ks2 = k0 ^ k1 ^ _PARITY
    inj = [(k1, (ks2 + 1) & _M32), (ks2, (k0 + 2) & _M32),
           (k0, (k1 + 3) & _M32), (k1, (ks2 + 4) & _M32),
           (ks2, (k0 + 5) & _M32)]
    a = (x0 + k0) & _M32
    b = (x1 + k1) & _M32
    for blk in range(5):
        for r in _ROTS[blk % 2]:
            a = (a + b) & _M32
            b = (((b << r) & _M32) | (b >> (32 - r))) ^ a
        pa, pb = inj[blk]
        a = (a + pa) & _M32
        b = (b + pb) & _M32
    return a, b


def _split_host(k0, k1):
    a0, b0 = _tf_host(k0, k1, 0, 0)
    a1, b1 = _tf_host(k0, k1, 0, 1)
    return (a0, b0), (a1, b1)


_KU, _KR = _split_host(0, 42)
_K1R, _K2R = _split_host(*_KR)
_KEYS = (_KU, _K1R, _K2R)


def _inj_consts(k0, k1):
    ks2 = k0 ^ k1 ^ _PARITY
    return [(k1, (ks2 + 1) & _M32), (ks2, (k0 + 2) & _M32),
            (k0, (k1 + 3) & _M32), (k1, (ks2 + 4) & _M32),
            (ks2, (k0 + 5) & _M32)]


_INJ = tuple(_inj_consts(k0, k1) for (k0, k1) in _KEYS)


def _tf3_bits(xlo):
    a = [jnp.full((_LANES,), np.uint32(k0), jnp.uint32) for (k0, _) in _KEYS]
    b = [xlo + np.uint32(k1) for (_, k1) in _KEYS]
    for blk in range(5):
        for r in _ROTS[blk % 2]:
            for t in range(3):
                a[t] = a[t] + b[t]
            for t in range(3):
                b[t] = ((b[t] << r) | (b[t] >> (32 - r))) ^ a[t]
        for t in range(3):
            pa, pb = _INJ[t][blk]
            a[t] = a[t] + np.uint32(pa)
            b[t] = b[t] + np.uint32(pb)
    return [a[t] ^ b[t] for t in range(3)]


_NC, _NS = 2, 16
_NW = _NC * _NS
_PER = _N // _NW
_CHUNKS = _PER // _LANES


def _rr_body(arr_hbm, ps_hbm, out_hbm, arr_v, ps_v, out_v, sem_in, sem_ps):
    wid = lax.axis_index("s") * _NC + lax.axis_index("c")
    base = wid * _PER
    cp_arr = pltpu.async_copy(arr_hbm.at[pl.ds(base, _PER)], arr_v, sem_in)
    cp_ps = pltpu.async_copy(ps_hbm, ps_v.at[pl.ds(0, 10)], sem_ps)
    cp_arr.wait()
    cp_ps.wait()
    ps_vec = ps_v[...]

    def body(j, carry):
        off = j * _LANES
        a_f = arr_v[pl.ds(off, _LANES)]
        idx = a_f.astype(jnp.int32)
        gidx = (lax.iota(jnp.int32, _LANES) + (base + off)).astype(jnp.uint32)
        bits_u, bits_h, bits_l = _tf3_bits(gidx)
        fbits = (bits_u >> np.uint32(9)) | np.uint32(0x3F800000)
        u = lax.bitcast_convert_type(fbits, jnp.float32) - jnp.float32(1.0)
        nine = np.uint32(9)
        ro = ((bits_h % nine) * np.uint32(4) + (bits_l % nine)) % nine
        r_i = ro.astype(jnp.int32)
        resp = jnp.where(r_i >= idx, r_i + jnp.int32(1), r_i)
        p_elem = jnp.take_along_axis(ps_vec, idx, axis=0)
        keep = u > p_elem
        out_v[pl.ds(off, _LANES)] = jnp.where(keep, a_f,
                                              resp.astype(jnp.float32))
        return carry

    lax.fori_loop(0, _CHUNKS, body, 0)
    pltpu.sync_copy(out_v, out_hbm.at[pl.ds(base, _PER)])


@functools.cache
def _build():
    mesh = plsc.VectorSubcoreMesh(core_axis_name="c", subcore_axis_name="s",
                                  num_cores=_NC, num_subcores=_NS)
    return pl.kernel(
        _rr_body,
        out_type=jax.ShapeDtypeStruct((_N,), jnp.float32),
        mesh=mesh,
        scratch_types=[
            pltpu.VMEM((_PER,), jnp.float32),
            pltpu.VMEM((_LANES,), jnp.float32),
            pltpu.VMEM((_PER,), jnp.float32),
            pltpu.SemaphoreType.DMA,
            pltpu.SemaphoreType.DMA,
        ],
    )


def kernel(arr, ps):
    return _build()(arr, ps)
